# Initial kernel scaffold; baseline (speedup 1.0000x reference)
import jax
import jax.numpy as jnp
from jax.experimental import pallas as pl


def kernel(x, embed):
    raise NotImplementedError("write your pallas kernel here")


# speedup vs baseline: 1.7657x; 1.7657x over previous
import functools

import jax
import jax.numpy as jnp
from jax import lax
from jax.experimental import pallas as pl
from jax.experimental.pallas import tpu as pltpu
from jax.experimental.pallas import tpu_sc as plsc

_DIM = 64
_C = 1024
_TM = 512


def _dist_kernel(x_ref, e_ref, idx_ref):
    x = x_ref[...]
    e = e_ref[...]
    cross = lax.dot_general(
        x, e, (((1,), (1,)), ((), ())), preferred_element_type=jnp.float32
    )
    x_sq = jnp.sum(x * x, axis=1, keepdims=True)
    e_sq = jnp.sum(e * e, axis=1)[None, :]
    sq = jnp.clip(x_sq - 2.0 * cross + e_sq, 0.0, None)
    d = jnp.sqrt(sq)
    dmin = jnp.min(d, axis=1, keepdims=True)
    c_iota = lax.broadcasted_iota(jnp.int32, d.shape, 1)
    idx = jnp.min(jnp.where(d == dmin, c_iota, jnp.int32(_C)), axis=1)
    idx_ref[...] = idx


def _nearest_indices(xf, e):
    m = xf.shape[0]
    return pl.pallas_call(
        _dist_kernel,
        grid=(m // _TM,),
        in_specs=[
            pl.BlockSpec((_TM, _DIM), lambda i: (i, 0)),
            pl.BlockSpec((_C, _DIM), lambda i: (0, 0)),
        ],
        out_specs=pl.BlockSpec((_TM,), lambda i: (i,)),
        out_shape=jax.ShapeDtypeStruct((m,), jnp.int32),
    )(xf, e)


def _make_sc_gather(m):
    info = plsc.get_sparse_core_info()
    nc, ns = info.num_cores, info.num_subcores
    nw = nc * ns
    assert m % (8 * nw) == 0
    bpw = m // nw
    ch = 128
    assert bpw % ch == 0
    mesh = plsc.VectorSubcoreMesh(core_axis_name="c", subcore_axis_name="s")

    @functools.partial(
        pl.kernel,
        mesh=mesh,
        compiler_params=pltpu.CompilerParams(use_tc_tiling_on_sc=False),
        out_type=jax.ShapeDtypeStruct((m, _DIM), jnp.float32),
        scratch_types=[
            pltpu.VMEM((bpw,), jnp.int32),
            pltpu.VMEM((bpw, _DIM), jnp.float32),
            pltpu.SemaphoreType.DMA,
        ],
    )
    def gather(table_hbm, idx_hbm, out_hbm, idx_v, rows_v, sem):
        wid = lax.axis_index("s") * nc + lax.axis_index("c")
        base = wid * bpw
        pltpu.sync_copy(idx_hbm.at[pl.ds(base, bpw)], idx_v)
        copies = [
            pltpu.async_copy(
                table_hbm.at[idx_v.at[pl.ds(j * ch, ch)]],
                rows_v.at[pl.ds(j * ch, ch)],
                sem,
            )
            for j in range(bpw // ch)
        ]
        for c in copies:
            c.wait()
        pltpu.sync_copy(rows_v, out_hbm.at[pl.ds(base, bpw)])

    return gather


def kernel(x, embed):
    b, n, d = x.shape
    e = embed[0]
    xf = x.reshape(b * n, d)
    idx = _nearest_indices(xf, e)
    quant = _make_sc_gather(b * n)(e, idx)
    return quant.reshape(b, n, d), idx.reshape(b, n)


# speedup vs baseline: 2.2410x; 1.2692x over previous
import functools

import jax
import jax.numpy as jnp
from jax import lax
from jax.experimental import pallas as pl
from jax.experimental.pallas import tpu as pltpu
from jax.experimental.pallas import tpu_sc as plsc

_DIM = 64
_C = 1024
_TM = 512


def _dist_kernel(x_ref, xsq_ref, e2_ref, esq_ref, idx_ref):
    x = x_ref[...]
    e2 = e2_ref[...]
    neg2cross = lax.dot_general(
        x, e2, (((1,), (1,)), ((), ())), preferred_element_type=jnp.float32
    )
    x_sq = xsq_ref[...]
    e_sq = esq_ref[...]
    lane = lax.broadcasted_iota(jnp.int32, (x.shape[0], 128), 1).astype(jnp.float32)
    run_s = jnp.full((x.shape[0], 128), jnp.inf, jnp.float32)
    run_i = jnp.zeros((x.shape[0], 128), jnp.float32)
    for j in range(_C // 128):
        sl = neg2cross[:, j * 128 : (j + 1) * 128]
        esl = e_sq[:, j * 128 : (j + 1) * 128]
        s = (x_sq + sl) + esl
        better = s < run_s
        run_s = jnp.where(better, s, run_s)
        run_i = jnp.where(better, lane + jnp.float32(j * 128), run_i)
    d = jnp.sqrt(jnp.clip(run_s, 0.0, None))
    dmin = jnp.min(d, axis=1, keepdims=True)
    idx = jnp.min(jnp.where(d == dmin, run_i, jnp.float32(_C)), axis=1)
    idx_ref[...] = idx.astype(jnp.int32)


def _nearest_indices(xf, e):
    m = xf.shape[0]
    e2 = -2.0 * e
    esq = jnp.sum(e * e, axis=1)[None, :]
    xsq = jnp.sum(xf * xf, axis=-1, keepdims=True)
    return pl.pallas_call(
        _dist_kernel,
        grid=(m // _TM,),
        in_specs=[
            pl.BlockSpec((_TM, _DIM), lambda i: (i, 0)),
            pl.BlockSpec((_TM, 1), lambda i: (i, 0)),
            pl.BlockSpec((_C, _DIM), lambda i: (0, 0)),
            pl.BlockSpec((1, _C), lambda i: (0, 0)),
        ],
        out_specs=pl.BlockSpec((_TM,), lambda i: (i,)),
        out_shape=jax.ShapeDtypeStruct((m,), jnp.int32),
    )(xf, xsq, e2, esq)


def _make_sc_gather(m):
    info = plsc.get_sparse_core_info()
    nc, ns = info.num_cores, info.num_subcores
    nw = nc * ns
    assert m % (8 * nw) == 0
    bpw = m // nw
    ch = 128
    assert bpw % ch == 0
    mesh = plsc.VectorSubcoreMesh(core_axis_name="c", subcore_axis_name="s")

    @functools.partial(
        pl.kernel,
        mesh=mesh,
        compiler_params=pltpu.CompilerParams(use_tc_tiling_on_sc=False),
        out_type=jax.ShapeDtypeStruct((m, _DIM), jnp.float32),
        scratch_types=[
            pltpu.VMEM((bpw,), jnp.int32),
            pltpu.VMEM((bpw, _DIM), jnp.float32),
            pltpu.SemaphoreType.DMA,
        ],
    )
    def gather(table_hbm, idx_hbm, out_hbm, idx_v, rows_v, sem):
        wid = lax.axis_index("s") * nc + lax.axis_index("c")
        base = wid * bpw
        pltpu.sync_copy(idx_hbm.at[pl.ds(base, bpw)], idx_v)
        copies = [
            pltpu.async_copy(
                table_hbm.at[idx_v.at[pl.ds(j * ch, ch)]],
                rows_v.at[pl.ds(j * ch, ch)],
                sem,
            )
            for j in range(bpw // ch)
        ]
        for c in copies:
            c.wait()
        pltpu.sync_copy(rows_v, out_hbm.at[pl.ds(base, bpw)])

    return gather


def kernel(x, embed):
    b, n, d = x.shape
    e = embed[0]
    xf = x.reshape(b * n, d)
    idx = _nearest_indices(xf, e)
    quant = _make_sc_gather(b * n)(e, idx)
    return quant.reshape(b, n, d), idx.reshape(b, n)


# speedup vs baseline: 3.0881x; 1.3780x over previous
import functools

import jax
import jax.numpy as jnp
from jax import lax
from jax.experimental import pallas as pl
from jax.experimental.pallas import tpu as pltpu
from jax.experimental.pallas import tpu_sc as plsc

_DIM = 64
_C = 1024


def _dist_kernel(xt_ref, xsq_ref, e2_ref, esqb_ref, idx_ref):
    xt = xt_ref[0]
    e2 = e2_ref[...]
    neg2cross = lax.dot_general(
        e2, xt, (((1,), (0,)), ((), ())), preferred_element_type=jnp.float32
    )
    x_sq = xsq_ref[0]
    n = xt.shape[1]
    subl = lax.broadcasted_iota(jnp.int32, (8, n), 0).astype(jnp.float32)
    run_s = jnp.full((8, n), jnp.inf, jnp.float32)
    run_i = jnp.zeros((8, n), jnp.float32)
    for j in range(_C // 8):
        sl = neg2cross[j * 8 : (j + 1) * 8, :]
        esl = esqb_ref[pl.ds(j * 8, 8), :]
        s = (x_sq + sl) + esl
        better = s < run_s
        run_s = jnp.where(better, s, run_s)
        run_i = jnp.where(better, subl + jnp.float32(j * 8), run_i)
    d = jnp.sqrt(jnp.clip(run_s, 0.0, None))
    dmin = jnp.min(d, axis=0, keepdims=True)
    idx = jnp.min(jnp.where(d == dmin, run_i, jnp.float32(_C)), axis=0)
    idx_ref[0, 0, :] = idx.astype(jnp.int32)


def _nearest_indices(x, e):
    b, n, dim = x.shape
    xt = jnp.transpose(x, (0, 2, 1))
    xsq = jnp.sum(x * x, axis=-1)[:, None, :]
    e2 = -2.0 * e
    esq = jnp.sum(e * e, axis=1)
    esqb = jnp.broadcast_to(esq[:, None], (_C, n))
    idx3 = pl.pallas_call(
        _dist_kernel,
        grid=(b,),
        in_specs=[
            pl.BlockSpec((1, dim, n), lambda i: (i, 0, 0)),
            pl.BlockSpec((1, 1, n), lambda i: (i, 0, 0)),
            pl.BlockSpec((_C, dim), lambda i: (0, 0)),
            pl.BlockSpec((_C, n), lambda i: (0, 0)),
        ],
        out_specs=pl.BlockSpec((1, 1, n), lambda i: (i, 0, 0)),
        out_shape=jax.ShapeDtypeStruct((b, 1, n), jnp.int32),
    )(xt, xsq, e2, esqb)
    return idx3.reshape(b, n)


def _make_sc_gather(b, n):
    m = b * n
    info = plsc.get_sparse_core_info()
    nc, ns = info.num_cores, info.num_subcores
    nw = nc * ns
    assert m % (8 * nw) == 0
    bpw = m // nw
    rows_per_w = bpw // n
    assert rows_per_w * n == bpw
    ch = 128
    assert bpw % ch == 0
    mesh = plsc.VectorSubcoreMesh(core_axis_name="c", subcore_axis_name="s")

    @functools.partial(
        pl.kernel,
        mesh=mesh,
        compiler_params=pltpu.CompilerParams(use_tc_tiling_on_sc=False),
        out_type=jax.ShapeDtypeStruct((b, n, _DIM), jnp.float32),
        scratch_types=[
            pltpu.VMEM((bpw,), jnp.int32),
            pltpu.VMEM((bpw, _DIM), jnp.float32),
            pltpu.SemaphoreType.DMA,
        ],
    )
    def gather(table_hbm, idx_hbm, out_hbm, idx_v, rows_v, sem):
        wid = lax.axis_index("s") * nc + lax.axis_index("c")
        base = wid * bpw
        pltpu.sync_copy(idx_hbm.at[pl.ds(base, bpw)], idx_v)
        copies = [
            pltpu.async_copy(
                table_hbm.at[idx_v.at[pl.ds(j * ch, ch)]],
                rows_v.at[pl.ds(j * ch, ch)],
                sem,
            )
            for j in range(bpw // ch)
        ]
        for c in copies:
            c.wait()
        for r in range(rows_per_w):
            pltpu.sync_copy(
                rows_v.at[pl.ds(r * n, n)], out_hbm.at[wid * rows_per_w + r]
            )

    return gather


def kernel(x, embed):
    b, n, d = x.shape
    e = embed[0]
    idx = _nearest_indices(x, e)
    quant = _make_sc_gather(b, n)(e, idx.reshape(b * n))
    return quant, idx


# speedup vs baseline: 3.0884x; 1.0001x over previous
import functools

import jax
import jax.numpy as jnp
from jax import lax
from jax.experimental import pallas as pl
from jax.experimental.pallas import tpu as pltpu
from jax.experimental.pallas import tpu_sc as plsc

_DIM = 64
_C = 1024


def _dist_kernel(xt_ref, xsq_ref, e2_ref, esqb_ref, idx_ref):
    xt = xt_ref[0]
    e2 = e2_ref[...]
    neg2cross = lax.dot_general(
        e2, xt, (((1,), (0,)), ((), ())), preferred_element_type=jnp.float32
    )
    x_sq = xsq_ref[0]
    n = xt.shape[1]
    subl = lax.broadcasted_iota(jnp.int32, (8, n), 0).astype(jnp.float32)
    run_s = jnp.full((8, n), jnp.inf, jnp.float32)
    run_i = jnp.zeros((8, n), jnp.float32)
    for j in range(_C // 8):
        sl = neg2cross[j * 8 : (j + 1) * 8, :]
        esl = esqb_ref[pl.ds(j * 8, 8), :]
        s = (x_sq + sl) + esl
        better = s < run_s
        run_s = jnp.where(better, s, run_s)
        run_i = jnp.where(better, subl + jnp.float32(j * 8), run_i)
    d = jnp.sqrt(jnp.clip(run_s, 0.0, None))
    dmin = jnp.min(d, axis=0, keepdims=True)
    idx = jnp.min(jnp.where(d == dmin, run_i, jnp.float32(_C)), axis=0)
    idx_ref[0, 0, :] = idx.astype(jnp.int32)


def _nearest_indices(x, e):
    b, n, dim = x.shape
    xt = jnp.transpose(x, (0, 2, 1))
    xsq = jnp.sum(x * x, axis=-1)[:, None, :]
    e2 = -2.0 * e
    esq = jnp.sum(e * e, axis=1)
    esqb = jnp.broadcast_to(esq[:, None], (_C, n))
    idx3 = pl.pallas_call(
        _dist_kernel,
        grid=(b,),
        in_specs=[
            pl.BlockSpec((1, dim, n), lambda i: (i, 0, 0)),
            pl.BlockSpec((1, 1, n), lambda i: (i, 0, 0)),
            pl.BlockSpec((_C, dim), lambda i: (0, 0)),
            pl.BlockSpec((_C, n), lambda i: (0, 0)),
        ],
        out_specs=pl.BlockSpec((1, 1, n), lambda i: (i, 0, 0)),
        out_shape=jax.ShapeDtypeStruct((b, 1, n), jnp.int32),
        compiler_params=pltpu.CompilerParams(
            dimension_semantics=("parallel",)
        ),
    )(xt, xsq, e2, esqb)
    return idx3.reshape(b, n)


def _make_sc_gather(b, n):
    m = b * n
    info = plsc.get_sparse_core_info()
    nc, ns = info.num_cores, info.num_subcores
    nw = nc * ns
    assert m % (8 * nw) == 0
    bpw = m // nw
    rows_per_w = bpw // n
    assert rows_per_w * n == bpw
    ch = 128
    assert bpw % ch == 0
    mesh = plsc.VectorSubcoreMesh(core_axis_name="c", subcore_axis_name="s")

    @functools.partial(
        pl.kernel,
        mesh=mesh,
        compiler_params=pltpu.CompilerParams(use_tc_tiling_on_sc=False),
        out_type=jax.ShapeDtypeStruct((b, n, _DIM), jnp.float32),
        scratch_types=[
            pltpu.VMEM((bpw,), jnp.int32),
            pltpu.VMEM((bpw, _DIM), jnp.float32),
            pltpu.SemaphoreType.DMA,
        ],
    )
    def gather(table_hbm, idx_hbm, out_hbm, idx_v, rows_v, sem):
        wid = lax.axis_index("s") * nc + lax.axis_index("c")
        base = wid * bpw
        pltpu.sync_copy(idx_hbm.at[pl.ds(base, bpw)], idx_v)
        copies = [
            pltpu.async_copy(
                table_hbm.at[idx_v.at[pl.ds(j * ch, ch)]],
                rows_v.at[pl.ds(j * ch, ch)],
                sem,
            )
            for j in range(bpw // ch)
        ]
        for c in copies:
            c.wait()
        for r in range(rows_per_w):
            pltpu.sync_copy(
                rows_v.at[pl.ds(r * n, n)], out_hbm.at[wid * rows_per_w + r]
            )

    return gather


def kernel(x, embed):
    b, n, d = x.shape
    e = embed[0]
    idx = _nearest_indices(x, e)
    quant = _make_sc_gather(b, n)(e, idx.reshape(b * n))
    return quant, idx


# speedup vs baseline: 3.4657x; 1.1222x over previous
import functools

import jax
import jax.numpy as jnp
from jax import lax
from jax.experimental import pallas as pl
from jax.experimental.pallas import tpu as pltpu
from jax.experimental.pallas import tpu_sc as plsc

_DIM = 64
_C = 1024


def _dist_kernel(xt_ref, xsq_ref, e2_ref, esqb_ref, idx_ref):
    e2 = e2_ref[...]
    for r in range(xt_ref.shape[0]):
        xt = xt_ref[r]
        neg2cross = lax.dot_general(
            e2, xt, (((1,), (0,)), ((), ())), preferred_element_type=jnp.float32
        )
        x_sq = xsq_ref[r]
        n = xt.shape[1]
        subl = lax.broadcasted_iota(jnp.int32, (8, n), 0).astype(jnp.float32)
        run_s = jnp.full((8, n), jnp.inf, jnp.float32)
        run_i = jnp.zeros((8, n), jnp.float32)
        for j in range(_C // 8):
            sl = neg2cross[j * 8 : (j + 1) * 8, :]
            esl = esqb_ref[pl.ds(j * 8, 8), :]
            s = (x_sq + sl) + esl
            better = s < run_s
            run_s = jnp.where(better, s, run_s)
            run_i = jnp.where(better, subl + jnp.float32(j * 8), run_i)
        d = jnp.sqrt(jnp.clip(run_s, 0.0, None))
        dmin = jnp.min(d, axis=0, keepdims=True)
        idx = jnp.min(jnp.where(d == dmin, run_i, jnp.float32(_C)), axis=0)
        idx_ref[r, 0, :] = idx.astype(jnp.int32)


def _nearest_indices(x, e):
    b, n, dim = x.shape
    xt = jnp.transpose(x, (0, 2, 1))
    xsq = jnp.sum(x * x, axis=-1)[:, None, :]
    e2 = -2.0 * e
    esq = jnp.sum(e * e, axis=1)
    esqb = jnp.broadcast_to(esq[:, None], (_C, n))
    rb = 2
    idx3 = pl.pallas_call(
        _dist_kernel,
        grid=(b // rb,),
        in_specs=[
            pl.BlockSpec((rb, dim, n), lambda i: (i, 0, 0)),
            pl.BlockSpec((rb, 1, n), lambda i: (i, 0, 0)),
            pl.BlockSpec((_C, dim), lambda i: (0, 0)),
            pl.BlockSpec((_C, n), lambda i: (0, 0)),
        ],
        out_specs=pl.BlockSpec((rb, 1, n), lambda i: (i, 0, 0)),
        out_shape=jax.ShapeDtypeStruct((b, 1, n), jnp.int32),
        compiler_params=pltpu.CompilerParams(
            dimension_semantics=("parallel",)
        ),
    )(xt, xsq, e2, esqb)
    return idx3.reshape(b, n)


def _make_sc_gather(b, n):
    m = b * n
    info = plsc.get_sparse_core_info()
    nc, ns = info.num_cores, info.num_subcores
    nw = nc * ns
    assert m % (8 * nw) == 0
    bpw = m // nw
    rows_per_w = bpw // n
    assert rows_per_w * n == bpw
    ch = 128
    assert bpw % ch == 0
    mesh = plsc.VectorSubcoreMesh(core_axis_name="c", subcore_axis_name="s")

    @functools.partial(
        pl.kernel,
        mesh=mesh,
        compiler_params=pltpu.CompilerParams(use_tc_tiling_on_sc=False),
        out_type=jax.ShapeDtypeStruct((b, n, _DIM), jnp.float32),
        scratch_types=[
            pltpu.VMEM((bpw,), jnp.int32),
            pltpu.VMEM((bpw, _DIM), jnp.float32),
            pltpu.SemaphoreType.DMA,
        ],
    )
    def gather(table_hbm, idx_hbm, out_hbm, idx_v, rows_v, sem):
        wid = lax.axis_index("s") * nc + lax.axis_index("c")
        base = wid * bpw
        pltpu.sync_copy(idx_hbm.at[pl.ds(base, bpw)], idx_v)
        copies = [
            pltpu.async_copy(
                table_hbm.at[idx_v.at[pl.ds(j * ch, ch)]],
                rows_v.at[pl.ds(j * ch, ch)],
                sem,
            )
            for j in range(bpw // ch)
        ]
        for c in copies:
            c.wait()
        for r in range(rows_per_w):
            pltpu.sync_copy(
                rows_v.at[pl.ds(r * n, n)], out_hbm.at[wid * rows_per_w + r]
            )

    return gather


def kernel(x, embed):
    b, n, d = x.shape
    e = embed[0]
    idx = _nearest_indices(x, e)
    quant = _make_sc_gather(b, n)(e, idx.reshape(b * n))
    return quant, idx


# speedup vs baseline: 3.5704x; 1.0302x over previous
import functools

import jax
import jax.numpy as jnp
from jax import lax
from jax.experimental import pallas as pl
from jax.experimental.pallas import tpu as pltpu
from jax.experimental.pallas import tpu_sc as plsc

_DIM = 64
_C = 1024


def _dist_kernel(xt_ref, xsq_ref, e2_ref, esqb_ref, idx_ref):
    e2 = e2_ref[...]
    for r in range(xt_ref.shape[0]):
        xt = xt_ref[r]
        neg2cross = lax.dot_general(
            e2, xt, (((1,), (0,)), ((), ())), preferred_element_type=jnp.float32
        )
        x_sq = xsq_ref[r]
        n = xt.shape[1]
        subl = lax.broadcasted_iota(jnp.int32, (8, n), 0).astype(jnp.float32)
        run_s = jnp.full((8, n), jnp.inf, jnp.float32)
        run_i = jnp.zeros((8, n), jnp.float32)
        for j in range(_C // 8):
            sl = neg2cross[j * 8 : (j + 1) * 8, :]
            esl = esqb_ref[pl.ds(j * 8, 8), :]
            s = (x_sq + sl) + esl
            better = s < run_s
            run_s = jnp.where(better, s, run_s)
            run_i = jnp.where(better, subl + jnp.float32(j * 8), run_i)
        d = jnp.sqrt(jnp.clip(run_s, 0.0, None))
        dmin = jnp.min(d, axis=0, keepdims=True)
        idx = jnp.min(jnp.where(d == dmin, run_i, jnp.float32(_C)), axis=0)
        idx_ref[r, 0, :] = idx.astype(jnp.int32)


def _nearest_indices(x, e):
    b, n, dim = x.shape
    xt = jnp.transpose(x, (0, 2, 1))
    xsq = jnp.sum(x * x, axis=-1)[:, None, :]
    e2 = -2.0 * e
    esq = jnp.sum(e * e, axis=1)
    esqb = jnp.broadcast_to(esq[:, None], (_C, n))
    rb = 4
    idx3 = pl.pallas_call(
        _dist_kernel,
        grid=(b // rb,),
        in_specs=[
            pl.BlockSpec((rb, dim, n), lambda i: (i, 0, 0)),
            pl.BlockSpec((rb, 1, n), lambda i: (i, 0, 0)),
            pl.BlockSpec((_C, dim), lambda i: (0, 0)),
            pl.BlockSpec((_C, n), lambda i: (0, 0)),
        ],
        out_specs=pl.BlockSpec((rb, 1, n), lambda i: (i, 0, 0)),
        out_shape=jax.ShapeDtypeStruct((b, 1, n), jnp.int32),
        compiler_params=pltpu.CompilerParams(
            dimension_semantics=("parallel",)
        ),
    )(xt, xsq, e2, esqb)
    return idx3.reshape(b, n)


def _make_sc_gather(b, n):
    m = b * n
    info = plsc.get_sparse_core_info()
    nc, ns = info.num_cores, info.num_subcores
    nw = nc * ns
    assert m % (8 * nw) == 0
    bpw = m // nw
    rows_per_w = bpw // n
    assert rows_per_w * n == bpw
    ch = 128
    assert bpw % ch == 0
    mesh = plsc.VectorSubcoreMesh(core_axis_name="c", subcore_axis_name="s")

    @functools.partial(
        pl.kernel,
        mesh=mesh,
        compiler_params=pltpu.CompilerParams(use_tc_tiling_on_sc=False),
        out_type=jax.ShapeDtypeStruct((b, n, _DIM), jnp.float32),
        scratch_types=[
            pltpu.VMEM((bpw,), jnp.int32),
            pltpu.VMEM((bpw, _DIM), jnp.float32),
            pltpu.SemaphoreType.DMA,
        ],
    )
    def gather(table_hbm, idx_hbm, out_hbm, idx_v, rows_v, sem):
        wid = lax.axis_index("s") * nc + lax.axis_index("c")
        base = wid * bpw
        pltpu.sync_copy(idx_hbm.at[pl.ds(base, bpw)], idx_v)
        copies = [
            pltpu.async_copy(
                table_hbm.at[idx_v.at[pl.ds(j * ch, ch)]],
                rows_v.at[pl.ds(j * ch, ch)],
                sem,
            )
            for j in range(bpw // ch)
        ]
        for c in copies:
            c.wait()
        for r in range(rows_per_w):
            pltpu.sync_copy(
                rows_v.at[pl.ds(r * n, n)], out_hbm.at[wid * rows_per_w + r]
            )

    return gather


def kernel(x, embed):
    b, n, d = x.shape
    e = embed[0]
    idx = _nearest_indices(x, e)
    quant = _make_sc_gather(b, n)(e, idx.reshape(b * n))
    return quant, idx
